# Initial kernel scaffold; baseline (speedup 1.0000x reference)
import jax
import jax.numpy as jnp
from jax.experimental import pallas as pl


def kernel(x, table):
    raise NotImplementedError("write your pallas kernel here")


# speedup vs baseline: 1.2285x; 1.2285x over previous
import functools
import math

import jax
import jax.numpy as jnp
from jax import lax
from jax.experimental import pallas as pl
from jax.experimental.pallas import tpu as pltpu
from jax.experimental.pallas import tpu_sc as plsc

_VOCAB = 100000
_D = 512
_B = 4
_S = 2048
_SCALAR = 10000.0

_NC = 2
_NS = 16
_NW = _NC * _NS
_CHUNK = _S // _NW
_LANES = 16


def _pos_body(o_ref):
    s = lax.broadcasted_iota(jnp.int32, (_S, _D), 0).astype(jnp.float32)
    j_int = lax.broadcasted_iota(jnp.int32, (_S, _D), 1)
    j = j_int.astype(jnp.float32)
    inv_freq = jnp.exp(j * (-2.0 * math.log(_SCALAR) / _D))
    pos = s * inv_freq
    even = (j_int & 1) == 0
    o_ref[...] = jnp.where(even, jnp.sin(pos), jnp.cos(pos))


_pos_table = pl.pallas_call(
    _pos_body,
    out_shape=jax.ShapeDtypeStruct((_S, _D), jnp.float32),
)


def _sc_embed_body(table_hbm, idx_hbm, pos_hbm, out_hbm,
                   idx_v, pos_v, r0, r1, sem0, sem1):
    wid = lax.axis_index("s") * _NC + lax.axis_index("c")
    s_base = wid * _CHUNK
    pltpu.sync_copy(idx_hbm.at[wid], idx_v)
    pltpu.sync_copy(pos_hbm.at[pl.ds(s_base, _CHUNK)], pos_v)

    bufs = (r0, r1)
    sems = (sem0, sem1)
    handles = [None, None]
    handles[0] = pltpu.async_copy(table_hbm.at[idx_v.at[0]], r0, sem0)
    for b in range(_B):
        rv = bufs[b % 2]
        if b + 1 < _B:
            handles[(b + 1) % 2] = pltpu.async_copy(
                table_hbm.at[idx_v.at[b + 1]], bufs[(b + 1) % 2],
                sems[(b + 1) % 2])
        handles[b % 2].wait()

        def _row_add(i, _, rv=rv):
            for j in range(_D // _LANES):
                sl = pl.ds(j * _LANES, _LANES)
                rv[i, sl] = rv[i, sl] + pos_v[i, sl]
            return 0

        lax.fori_loop(0, _CHUNK, _row_add, 0)
        pltpu.sync_copy(rv, out_hbm.at[pl.ds(b * _S + s_base, _CHUNK)])


@functools.lru_cache(maxsize=None)
def _get_sc_embed():
    return functools.partial(
        pl.kernel,
        mesh=plsc.VectorSubcoreMesh(core_axis_name="c", subcore_axis_name="s"),
        out_type=jax.ShapeDtypeStruct((_B * _S, _D), jnp.float32),
        scratch_types=[
            pltpu.VMEM((_B, _CHUNK), jnp.int32),
            pltpu.VMEM((_CHUNK, _D), jnp.float32),
            pltpu.VMEM((_CHUNK, _D), jnp.float32),
            pltpu.VMEM((_CHUNK, _D), jnp.float32),
            pltpu.SemaphoreType.DMA,
            pltpu.SemaphoreType.DMA,
        ],
    )(_sc_embed_body)


def kernel(x, table):
    _sc_embed = _get_sc_embed()
    pos = _pos_table()
    idx = x.astype(jnp.int32).reshape(_B, _NW, _CHUNK).transpose(1, 0, 2)
    out = _sc_embed(table, idx, pos)
    return out.reshape(_B, _S, _D)
